# baseline (speedup 1.0000x reference)
import functools

import jax
import jax.numpy as jnp
from jax.experimental import pallas as pl

B, N, C, K = 2, 2048, 1024, 32


def _fc_body(x_ref, wt_ref, bias_ref, feat_ref, pos_ref):
    fp = jax.lax.dot_general(
        x_ref[...], wt_ref[...], (((1,), (1,)), ((), ())),
        preferred_element_type=jnp.float32,
    ) + bias_ref[...]
    feat_ref[...] = fp[:, :C].astype(jnp.bfloat16)
    pr = fp[:, C:]
    nrm = jnp.sqrt(jnp.sum(pr * pr, axis=1, keepdims=True))
    pos_ref[...] = (pr / jnp.maximum(nrm, 1e-12)).astype(jnp.bfloat16)


def _attn_body(posb_ref, posf_ref, feat_ref, out_ref, *, bm: int):
    pb = posb_ref[0]
    pf = posf_ref[0]
    sim = jax.lax.dot_general(
        pb, pf, (((1,), (1,)), ((), ())),
        preferred_element_type=jnp.float32,
    )

    hm = bm // 4
    nch = N // 128

    def pass2(_, s):
        s4 = s.reshape(hm, nch // 2, 2, 128)
        a, b = s4[:, :, 0], s4[:, :, 1]
        cur1, cur2 = jnp.maximum(a, b), jnp.minimum(a, b)
        while cur1.shape[1] > 1:
            c1 = cur1.reshape(hm, -1, 2, 128)
            c2 = cur2.reshape(hm, -1, 2, 128)
            a1, b1 = c1[:, :, 0], c1[:, :, 1]
            a2, b2 = c2[:, :, 0], c2[:, :, 1]
            cur1 = jnp.maximum(a1, b1)
            cur2 = jnp.maximum(jnp.minimum(a1, b1), jnp.maximum(a2, b2))
        f1, f2 = cur1[:, 0], cur2[:, 0]
        m1 = jnp.max(f1, axis=1, keepdims=True)
        cand = jnp.where(f1 == m1, f2, f1)
        m2 = jnp.max(cand, axis=1, keepdims=True)
        return jnp.where(s >= m2, -jnp.inf, s)

    for h in range(4):
        sh = sim[h * hm:(h + 1) * hm]
        s_fin = jax.lax.fori_loop(0, K // 2, pass2, sh, unroll=K // 2)
        e = jnp.where(s_fin == -jnp.inf, jnp.exp(sh), 0.0)
        r = 1.0 / jnp.sum(e, axis=1, keepdims=True)
        o = jax.lax.dot_general(
            e, feat_ref[0], (((1,), (0,)), ((), ())),
            preferred_element_type=jnp.float32,
        )
        out_ref[0, h * hm:(h + 1) * hm, :] = o * r


@jax.jit
def kernel(x, W, bias):
    bm = 512
    x2 = x.reshape(B * N, C)
    wt = W
    bias2 = bias.reshape(1, 2 * C)

    feat, pos = pl.pallas_call(
        _fc_body,
        grid=(B * N // bm,),
        in_specs=[
            pl.BlockSpec((bm, C), lambda i: (i, 0)),
            pl.BlockSpec((2 * C, C), lambda i: (0, 0)),
            pl.BlockSpec((1, 2 * C), lambda i: (0, 0)),
        ],
        out_specs=[
            pl.BlockSpec((bm, C), lambda i: (i, 0)),
            pl.BlockSpec((bm, C), lambda i: (i, 0)),
        ],
        out_shape=[
            jax.ShapeDtypeStruct((B * N, C), jnp.bfloat16),
            jax.ShapeDtypeStruct((B * N, C), jnp.bfloat16),
        ],
    )(x2, wt, bias2)

    feat3 = feat.reshape(B, N, C)
    pos3 = pos.reshape(B, N, C)

    out = pl.pallas_call(
        functools.partial(_attn_body, bm=bm),
        grid=(B, N // bm),
        in_specs=[
            pl.BlockSpec((1, bm, C), lambda b, i: (b, i, 0)),
            pl.BlockSpec((1, N, C), lambda b, i: (b, 0, 0)),
            pl.BlockSpec((1, N, C), lambda b, i: (b, 0, 0)),
        ],
        out_specs=pl.BlockSpec((1, bm, C), lambda b, i: (b, i, 0)),
        out_shape=jax.ShapeDtypeStruct((B, N, C), jnp.float32),
    )(pos3, pos3, feat3)

    return out


# speedup vs baseline: 6.8272x; 6.8272x over previous
import functools

import jax
import jax.numpy as jnp
from jax.experimental import pallas as pl

B, N, C, K = 2, 2048, 1024, 32


def _fc_body(x_ref, wt_ref, bias_ref, feat_ref, pos_ref):
    fp = jax.lax.dot_general(
        x_ref[...], wt_ref[...], (((1,), (1,)), ((), ())),
        preferred_element_type=jnp.float32,
    ) + bias_ref[...]
    feat_ref[...] = fp[:, :C].astype(jnp.bfloat16)
    pr = fp[:, C:]
    nrm = jnp.sqrt(jnp.sum(pr * pr, axis=1, keepdims=True))
    pos_ref[...] = (pr / jnp.maximum(nrm, 1e-12)).astype(jnp.bfloat16)


def _attn_body(posb_ref, posf_ref, feat_ref, out_ref, *, bm: int):
    pb = posb_ref[0]
    pf = posf_ref[0]
    sim = jax.lax.dot_general(
        pb, pf, (((1,), (1,)), ((), ())),
        preferred_element_type=jnp.float32,
    )

    hm = bm // 4

    def pass2(_, s):
        w = N // 2
        cur1 = jnp.maximum(s[:, :w], s[:, w:])
        cur2 = jnp.minimum(s[:, :w], s[:, w:])
        while cur1.shape[1] > 128:
            w2 = cur1.shape[1] // 2
            a1, b1 = cur1[:, :w2], cur1[:, w2:]
            a2, b2 = cur2[:, :w2], cur2[:, w2:]
            n2 = jnp.maximum(jnp.minimum(a1, b1), jnp.maximum(a2, b2))
            cur1, cur2 = jnp.maximum(a1, b1), n2
        m1 = jnp.max(cur1, axis=1, keepdims=True)
        cand = jnp.where(cur1 == m1, cur2, cur1)
        m2 = jnp.max(cand, axis=1, keepdims=True)
        return jnp.where(s >= m2, -jnp.inf, s)

    for h in range(4):
        sh = sim[h * hm:(h + 1) * hm]
        s_fin = jax.lax.fori_loop(0, K // 2, pass2, sh, unroll=K // 2)
        e = jnp.where(s_fin == -jnp.inf, jnp.exp(sh), 0.0)
        r = 1.0 / jnp.sum(e, axis=1, keepdims=True)
        o = jax.lax.dot_general(
            e, feat_ref[0], (((1,), (0,)), ((), ())),
            preferred_element_type=jnp.float32,
        )
        out_ref[0, h * hm:(h + 1) * hm, :] = o * r


@jax.jit
def kernel(x, W, bias):
    bm = 512
    x2 = x.reshape(B * N, C)
    wt = W
    bias2 = bias.reshape(1, 2 * C)

    feat, pos = pl.pallas_call(
        _fc_body,
        grid=(B * N // bm,),
        in_specs=[
            pl.BlockSpec((bm, C), lambda i: (i, 0)),
            pl.BlockSpec((2 * C, C), lambda i: (0, 0)),
            pl.BlockSpec((1, 2 * C), lambda i: (0, 0)),
        ],
        out_specs=[
            pl.BlockSpec((bm, C), lambda i: (i, 0)),
            pl.BlockSpec((bm, C), lambda i: (i, 0)),
        ],
        out_shape=[
            jax.ShapeDtypeStruct((B * N, C), jnp.bfloat16),
            jax.ShapeDtypeStruct((B * N, C), jnp.bfloat16),
        ],
    )(x2, wt, bias2)

    feat3 = feat.reshape(B, N, C)
    pos3 = pos.reshape(B, N, C)

    out = pl.pallas_call(
        functools.partial(_attn_body, bm=bm),
        grid=(B, N // bm),
        in_specs=[
            pl.BlockSpec((1, bm, C), lambda b, i: (b, i, 0)),
            pl.BlockSpec((1, N, C), lambda b, i: (b, 0, 0)),
            pl.BlockSpec((1, N, C), lambda b, i: (b, 0, 0)),
        ],
        out_specs=pl.BlockSpec((1, bm, C), lambda b, i: (b, i, 0)),
        out_shape=jax.ShapeDtypeStruct((B, N, C), jnp.float32),
    )(pos3, pos3, feat3)

    return out


# speedup vs baseline: 7.0343x; 1.0303x over previous
import functools

import jax
import jax.numpy as jnp
from jax.experimental import pallas as pl

B, N, C, K = 2, 2048, 1024, 32


def _fc_body(x_ref, wt_ref, bias_ref, feat_ref, pos_ref):
    fp = jax.lax.dot_general(
        x_ref[...], wt_ref[...], (((1,), (1,)), ((), ())),
        preferred_element_type=jnp.float32,
    ) + bias_ref[...]
    feat_ref[...] = fp[:, :C].astype(jnp.bfloat16)
    pr = fp[:, C:]
    nrm = jnp.sqrt(jnp.sum(pr * pr, axis=1, keepdims=True))
    pos_ref[...] = (pr / jnp.maximum(nrm, 1e-12)).astype(jnp.bfloat16)


def _attn_body(posb_ref, posf_ref, feat_ref, out_ref, *, bm: int):
    pb = posb_ref[0]
    pf = posf_ref[0]
    sim = jax.lax.dot_general(
        pb, pf, (((1,), (1,)), ((), ())),
        preferred_element_type=jnp.float32,
    )

    def step(_, s):
        m = jnp.max(s, axis=1, keepdims=True)
        return jnp.where(s == m, -jnp.inf, s)

    hm = bm // 4
    for h in range(4):
        sh = sim[h * hm:(h + 1) * hm]
        s_fin = jax.lax.fori_loop(0, K, step, sh, unroll=K)
        e = jnp.where(s_fin == -jnp.inf, jnp.exp(sh), 0.0)
        r = 1.0 / jnp.sum(e, axis=1, keepdims=True)
        o = jax.lax.dot_general(
            e, feat_ref[0], (((1,), (0,)), ((), ())),
            preferred_element_type=jnp.float32,
        )
        out_ref[0, h * hm:(h + 1) * hm, :] = o * r


@jax.jit
def kernel(x, W, bias):
    bm = 512
    x2 = x.reshape(B * N, C)
    wt = W
    bias2 = bias.reshape(1, 2 * C)

    feat, pos = pl.pallas_call(
        _fc_body,
        grid=(B * N // bm,),
        in_specs=[
            pl.BlockSpec((bm, C), lambda i: (i, 0)),
            pl.BlockSpec((2 * C, C), lambda i: (0, 0)),
            pl.BlockSpec((1, 2 * C), lambda i: (0, 0)),
        ],
        out_specs=[
            pl.BlockSpec((bm, C), lambda i: (i, 0)),
            pl.BlockSpec((bm, C), lambda i: (i, 0)),
        ],
        out_shape=[
            jax.ShapeDtypeStruct((B * N, C), jnp.bfloat16),
            jax.ShapeDtypeStruct((B * N, C), jnp.bfloat16),
        ],
    )(x2, wt, bias2)

    feat3 = feat.reshape(B, N, C)
    pos3 = pos.reshape(B, N, C)

    out = pl.pallas_call(
        functools.partial(_attn_body, bm=bm),
        grid=(B, N // bm),
        in_specs=[
            pl.BlockSpec((1, bm, C), lambda b, i: (b, i, 0)),
            pl.BlockSpec((1, N, C), lambda b, i: (b, 0, 0)),
            pl.BlockSpec((1, N, C), lambda b, i: (b, 0, 0)),
        ],
        out_specs=pl.BlockSpec((1, bm, C), lambda b, i: (b, i, 0)),
        out_shape=jax.ShapeDtypeStruct((B, N, C), jnp.float32),
    )(pos3, pos3, feat3)

    return out
